# Initial kernel scaffold; baseline (speedup 1.0000x reference)
import jax
import jax.numpy as jnp
from jax.experimental import pallas as pl


def kernel(y, f_y, neighbors_index, neighbors_row_splits, W1, b1, W2, b2):
    raise NotImplementedError("write your pallas kernel here")


# speedup vs baseline: 1.0609x; 1.0609x over previous
import functools
import math

import jax
import jax.numpy as jnp
from jax import lax
from jax.experimental import pallas as pl
from jax.experimental.pallas import tpu as pltpu
from jax.experimental.pallas import tpu_sc as plsc

NC = 2
NS = 16
NW = NC * NS

D_PAD = 16
DF = 32


def _sc_mesh():
    return plsc.VectorSubcoreMesh(
        core_axis_name="c", subcore_axis_name="s", num_cores=NC, num_subcores=NS
    )


_SC_PARAMS = pltpu.CompilerParams(use_tc_tiling_on_sc=False)


def _make_gather(n_chunk, ch):
    @functools.partial(
        pl.kernel,
        out_type=[
            jax.ShapeDtypeStruct((NW, n_chunk, ch, D_PAD), jnp.float32),
            jax.ShapeDtypeStruct((NW, n_chunk, ch, D_PAD), jnp.float32),
        ],
        mesh=_sc_mesh(),
        scratch_types=[
            pltpu.VMEM((ch,), jnp.int32),
            pltpu.VMEM((ch,), jnp.int32),
            pltpu.VMEM((ch, D_PAD), jnp.float32),
            pltpu.VMEM((ch, D_PAD), jnp.float32),
            pltpu.SemaphoreType.DMA,
            pltpu.SemaphoreType.DMA,
        ],
        compiler_params=_SC_PARAMS,
    )
    def gather_k(y16, idx3, seg3, rep_out, self_out,
                 idx_v, seg_v, rep_v, self_v, sem1, sem2):
        wid = lax.axis_index("s") * NC + lax.axis_index("c")

        @pl.loop(0, n_chunk)
        def _body(j):
            pltpu.sync_copy(idx3.at[wid, j], idx_v)
            pltpu.sync_copy(seg3.at[wid, j], seg_v)
            cp1 = pltpu.async_copy(y16.at[idx_v], rep_v, sem1)
            cp2 = pltpu.async_copy(y16.at[seg_v], self_v, sem2)
            cp1.wait()
            cp2.wait()
            pltpu.sync_copy(rep_v, rep_out.at[wid, j])
            pltpu.sync_copy(self_v, self_out.at[wid, j])

    return gather_k


def _erf(x):
    a1, a2, a3, a4, a5 = (
        0.254829592, -0.284496736, 1.421413741, -1.453152027, 1.061405429
    )
    p = 0.3275911
    ax = jnp.abs(x)
    t = 1.0 / (1.0 + p * ax)
    poly = ((((a5 * t + a4) * t + a3) * t + a2) * t + a1) * t
    y = 1.0 - poly * jnp.exp(-ax * ax)
    return jnp.sign(x) * y


def _make_mlp(e, bt):
    grid = e // bt

    def mlp_body(rep_ref, self_ref, w1a_ref, w1b_ref, b1_ref, w2_ref, b2_ref,
                 out_ref):
        h = jnp.dot(rep_ref[...], w1a_ref[...],
                    preferred_element_type=jnp.float32)
        h = h + jnp.dot(self_ref[...], w1b_ref[...],
                        preferred_element_type=jnp.float32)
        h = h + b1_ref[...]
        inv_sqrt2 = 0.7071067811865476
        h = 0.5 * h * (1.0 + _erf(h * inv_sqrt2))
        out_ref[...] = (
            jnp.dot(h, w2_ref[...], preferred_element_type=jnp.float32)
            + b2_ref[...]
        )

    return pl.pallas_call(
        mlp_body,
        grid=(grid,),
        in_specs=[
            pl.BlockSpec((bt, D_PAD), lambda i: (i, 0)),
            pl.BlockSpec((bt, D_PAD), lambda i: (i, 0)),
            pl.BlockSpec((D_PAD, 64), lambda i: (0, 0)),
            pl.BlockSpec((D_PAD, 64), lambda i: (0, 0)),
            pl.BlockSpec((1, 64), lambda i: (0, 0)),
            pl.BlockSpec((64, DF), lambda i: (0, 0)),
            pl.BlockSpec((1, DF), lambda i: (0, 0)),
        ],
        out_specs=pl.BlockSpec((bt, DF), lambda i: (i, 0)),
        out_shape=jax.ShapeDtypeStruct((e, DF), jnp.float32),
    )


def _make_scatter(n, n_chunk, ch):
    rows_per_tile = n // NS

    @functools.partial(
        pl.kernel,
        out_type=jax.ShapeDtypeStruct((NC, n, DF), jnp.float32),
        mesh=_sc_mesh(),
        scratch_types=[
            pltpu.VMEM((ch,), jnp.int32),
            pltpu.VMEM((ch,), jnp.int32),
            pltpu.VMEM((ch, DF), jnp.float32),
            pltpu.VMEM((ch, DF), jnp.float32),
            pltpu.VMEM_SHARED((n, DF), jnp.float32),
            pltpu.SemaphoreType.DMA,
        ],
        compiler_params=_SC_PARAMS,
    )
    def scatter_k(kout3, fy, idx3, seg3, zeros_hbm, out_hbm,
                  idx_v, seg_v, k_v, f_v, acc, sem):
        cid = lax.axis_index("c")
        sid = lax.axis_index("s")
        wid = sid * NC + cid

        pltpu.sync_copy(
            zeros_hbm.at[pl.ds(sid * rows_per_tile, rows_per_tile)],
            acc.at[pl.ds(sid * rows_per_tile, rows_per_tile)],
        )
        plsc.subcore_barrier()

        @pl.loop(0, n_chunk)
        def _body(j):
            pltpu.sync_copy(idx3.at[wid, j], idx_v)
            pltpu.sync_copy(seg3.at[wid, j], seg_v)
            pltpu.sync_copy(kout3.at[wid, j], k_v)
            pltpu.async_copy(fy.at[idx_v], f_v, sem).wait()
            for r in range(ch):
                for hh in range(DF // 16):
                    s = pl.ds(hh * 16, 16)
                    k_v[r, s] = k_v[r, s] * f_v[r, s]
            pltpu.sync_copy(k_v, acc.at[seg_v], add=True)

        plsc.subcore_barrier()
        pltpu.sync_copy(
            acc.at[pl.ds(sid * rows_per_tile, rows_per_tile)],
            out_hbm.at[cid, pl.ds(sid * rows_per_tile, rows_per_tile)],
        )

    return scatter_k


def _make_combine(n, bn):
    grid = n // bn

    def comb_body(p_ref, inv_ref, out_ref):
        out_ref[...] = (p_ref[0] + p_ref[1]) * inv_ref[...]

    return pl.pallas_call(
        comb_body,
        grid=(grid,),
        in_specs=[
            pl.BlockSpec((NC, bn, DF), lambda i: (0, i, 0)),
            pl.BlockSpec((bn, 1), lambda i: (i, 0)),
        ],
        out_specs=pl.BlockSpec((bn, DF), lambda i: (i, 0)),
        out_shape=jax.ShapeDtypeStruct((n, DF), jnp.float32),
    )


def kernel(y, f_y, neighbors_index, neighbors_row_splits, W1, b1, W2, b2):
    n, d_coord = y.shape
    e = neighbors_index.shape[0]
    assert e % NW == 0
    epw = e // NW
    ch = 128
    while epw % ch != 0 or ch % 8 != 0:
        ch -= 8
    n_chunk = epw // ch

    idx = neighbors_index.astype(jnp.int32)
    rs = neighbors_row_splits.astype(jnp.int32)
    seg = jnp.searchsorted(rs, jnp.arange(e, dtype=jnp.int32), side="right")
    seg = jnp.clip(seg.astype(jnp.int32) - 1, 0, n - 1)

    idx3 = idx.reshape(NW, n_chunk, ch)
    seg3 = seg.reshape(NW, n_chunk, ch)

    y16 = jnp.zeros((n, D_PAD), jnp.float32).at[:, :d_coord].set(y)

    rep16, self16 = _make_gather(n_chunk, ch)(y16, idx3, seg3)
    rep2 = rep16.reshape(e, D_PAD)
    self2 = self16.reshape(e, D_PAD)

    w1a = jnp.zeros((D_PAD, 64), jnp.float32).at[:d_coord].set(W1[:d_coord])
    w1b = jnp.zeros((D_PAD, 64), jnp.float32).at[:d_coord].set(W1[d_coord:])

    kout = _make_mlp(e, 4000)(rep2, self2, w1a, w1b,
                              b1.reshape(1, 64), W2, b2.reshape(1, DF))
    kout3 = kout.reshape(NW, n_chunk, ch, DF)

    zeros_hbm = jnp.zeros((n, DF), jnp.float32)
    partials = _make_scatter(n, n_chunk, ch)(kout3, f_y, idx3, seg3, zeros_hbm)

    counts = (rs[1:] - rs[:-1]).astype(jnp.float32)
    inv = (1.0 / jnp.maximum(counts, 1.0)).reshape(n, 1)

    bn = 2000
    while n % bn != 0:
        bn -= 8
    out = _make_combine(n, bn)(partials, inv)
    return out


# speedup vs baseline: 31.2713x; 29.4764x over previous
import functools
import math

import jax
import jax.numpy as jnp
from jax import lax
from jax.experimental import pallas as pl
from jax.experimental.pallas import tpu as pltpu
from jax.experimental.pallas import tpu_sc as plsc

NC = 2
NS = 16
NW = NC * NS

D_PAD = 16
DF = 32


def _sc_mesh():
    return plsc.VectorSubcoreMesh(
        core_axis_name="c", subcore_axis_name="s", num_cores=NC, num_subcores=NS
    )


_SC_PARAMS = pltpu.CompilerParams(use_tc_tiling_on_sc=False)


def _make_gather(n_chunk, ch):
    @functools.partial(
        pl.kernel,
        out_type=[
            jax.ShapeDtypeStruct((NW, n_chunk, ch, D_PAD), jnp.float32),
            jax.ShapeDtypeStruct((NW, n_chunk, ch, D_PAD), jnp.float32),
        ],
        mesh=_sc_mesh(),
        scratch_types=[
            pltpu.VMEM((ch,), jnp.int32),
            pltpu.VMEM((ch,), jnp.int32),
            pltpu.VMEM((ch, D_PAD), jnp.float32),
            pltpu.VMEM((ch, D_PAD), jnp.float32),
            pltpu.SemaphoreType.DMA,
            pltpu.SemaphoreType.DMA,
        ],
        compiler_params=_SC_PARAMS,
    )
    def gather_k(y16, idx3, seg3, rep_out, self_out,
                 idx_v, seg_v, rep_v, self_v, sem1, sem2):
        wid = lax.axis_index("s") * NC + lax.axis_index("c")

        @pl.loop(0, n_chunk)
        def _body(j):
            pltpu.sync_copy(idx3.at[wid, j], idx_v)
            pltpu.sync_copy(seg3.at[wid, j], seg_v)
            cp1 = pltpu.async_copy(y16.at[idx_v], rep_v, sem1)
            cp2 = pltpu.async_copy(y16.at[seg_v], self_v, sem2)
            cp1.wait()
            cp2.wait()
            pltpu.sync_copy(rep_v, rep_out.at[wid, j])
            pltpu.sync_copy(self_v, self_out.at[wid, j])

    return gather_k


def _erf(x):
    a1, a2, a3, a4, a5 = (
        0.254829592, -0.284496736, 1.421413741, -1.453152027, 1.061405429
    )
    p = 0.3275911
    ax = jnp.abs(x)
    t = 1.0 / (1.0 + p * ax)
    poly = ((((a5 * t + a4) * t + a3) * t + a2) * t + a1) * t
    y = 1.0 - poly * jnp.exp(-ax * ax)
    return jnp.sign(x) * y


def _make_mlp(e, bt):
    grid = e // bt

    def mlp_body(rep_ref, self_ref, w1a_ref, w1b_ref, b1_ref, w2_ref, b2_ref,
                 out_ref):
        h = jnp.dot(rep_ref[...], w1a_ref[...],
                    preferred_element_type=jnp.float32)
        h = h + jnp.dot(self_ref[...], w1b_ref[...],
                        preferred_element_type=jnp.float32)
        h = h + b1_ref[...]
        inv_sqrt2 = 0.7071067811865476
        h = 0.5 * h * (1.0 + _erf(h * inv_sqrt2))
        out_ref[...] = (
            jnp.dot(h, w2_ref[...], preferred_element_type=jnp.float32)
            + b2_ref[...]
        )

    return pl.pallas_call(
        mlp_body,
        grid=(grid,),
        in_specs=[
            pl.BlockSpec((bt, D_PAD), lambda i: (i, 0)),
            pl.BlockSpec((bt, D_PAD), lambda i: (i, 0)),
            pl.BlockSpec((D_PAD, 64), lambda i: (0, 0)),
            pl.BlockSpec((D_PAD, 64), lambda i: (0, 0)),
            pl.BlockSpec((1, 64), lambda i: (0, 0)),
            pl.BlockSpec((64, DF), lambda i: (0, 0)),
            pl.BlockSpec((1, DF), lambda i: (0, 0)),
        ],
        out_specs=pl.BlockSpec((bt, DF), lambda i: (i, 0)),
        out_shape=jax.ShapeDtypeStruct((e, DF), jnp.float32),
    )


def _make_scatter(n, n_chunk, ch):
    rows_per_tile = n // NS

    @functools.partial(
        pl.kernel,
        out_type=jax.ShapeDtypeStruct((NC, n, DF), jnp.float32),
        mesh=_sc_mesh(),
        scratch_types=[
            pltpu.VMEM((ch,), jnp.int32),
            pltpu.VMEM((ch,), jnp.int32),
            pltpu.VMEM((ch, DF), jnp.float32),
            pltpu.VMEM((ch, DF), jnp.float32),
            pltpu.VMEM_SHARED((n, DF), jnp.float32),
            pltpu.SemaphoreType.DMA,
        ],
        compiler_params=_SC_PARAMS,
    )
    def scatter_k(kout3, fy, idx3, seg3, zeros_hbm, out_hbm,
                  idx_v, seg_v, k_v, f_v, acc, sem):
        cid = lax.axis_index("c")
        sid = lax.axis_index("s")
        wid = sid * NC + cid

        pltpu.sync_copy(
            zeros_hbm.at[pl.ds(sid * rows_per_tile, rows_per_tile)],
            acc.at[pl.ds(sid * rows_per_tile, rows_per_tile)],
        )
        plsc.subcore_barrier()

        @pl.loop(0, n_chunk)
        def _body(j):
            pltpu.sync_copy(idx3.at[wid, j], idx_v)
            pltpu.sync_copy(seg3.at[wid, j], seg_v)
            pltpu.sync_copy(kout3.at[wid, j], k_v)
            pltpu.async_copy(fy.at[idx_v], f_v, sem).wait()
            for r in range(ch):
                for hh in range(DF // 16):
                    s = pl.ds(hh * 16, 16)
                    k_v[r, s] = k_v[r, s] * f_v[r, s]
            pltpu.sync_copy(k_v, acc.at[seg_v], add=True)

        plsc.subcore_barrier()
        pltpu.sync_copy(
            acc.at[pl.ds(sid * rows_per_tile, rows_per_tile)],
            out_hbm.at[cid, pl.ds(sid * rows_per_tile, rows_per_tile)],
        )

    return scatter_k


def _make_combine(n, bn):
    grid = n // bn

    def comb_body(p_ref, inv_ref, out_ref):
        out_ref[...] = (p_ref[0] + p_ref[1]) * inv_ref[...]

    return pl.pallas_call(
        comb_body,
        grid=(grid,),
        in_specs=[
            pl.BlockSpec((NC, bn, DF), lambda i: (0, i, 0)),
            pl.BlockSpec((bn, 1), lambda i: (i, 0)),
        ],
        out_specs=pl.BlockSpec((bn, DF), lambda i: (i, 0)),
        out_shape=jax.ShapeDtypeStruct((n, DF), jnp.float32),
    )


def kernel(y, f_y, neighbors_index, neighbors_row_splits, W1, b1, W2, b2):
    n, d_coord = y.shape
    e = neighbors_index.shape[0]
    assert e % NW == 0
    epw = e // NW
    ch = 128
    while epw % ch != 0 or ch % 8 != 0:
        ch -= 8
    n_chunk = epw // ch

    idx = neighbors_index.astype(jnp.int32)
    rs = neighbors_row_splits.astype(jnp.int32)
    hist = jnp.zeros((e,), jnp.int32).at[rs[1:n]].add(1, mode="drop")
    seg = jnp.clip(jnp.cumsum(hist), 0, n - 1)

    idx3 = idx.reshape(NW, n_chunk, ch)
    seg3 = seg.reshape(NW, n_chunk, ch)

    y16 = jnp.zeros((n, D_PAD), jnp.float32).at[:, :d_coord].set(y)

    rep16, self16 = _make_gather(n_chunk, ch)(y16, idx3, seg3)
    rep2 = rep16.reshape(e, D_PAD)
    self2 = self16.reshape(e, D_PAD)

    w1a = jnp.zeros((D_PAD, 64), jnp.float32).at[:d_coord].set(W1[:d_coord])
    w1b = jnp.zeros((D_PAD, 64), jnp.float32).at[:d_coord].set(W1[d_coord:])

    kout = _make_mlp(e, 4000)(rep2, self2, w1a, w1b,
                              b1.reshape(1, 64), W2, b2.reshape(1, DF))
    kout3 = kout.reshape(NW, n_chunk, ch, DF)

    zeros_hbm = jnp.zeros((n, DF), jnp.float32)
    partials = _make_scatter(n, n_chunk, ch)(kout3, f_y, idx3, seg3, zeros_hbm)

    counts = (rs[1:] - rs[:-1]).astype(jnp.float32)
    inv = (1.0 / jnp.maximum(counts, 1.0)).reshape(n, 1)

    bn = 2000
    while n % bn != 0:
        bn -= 8
    out = _make_combine(n, bn)(partials, inv)
    return out


# speedup vs baseline: 44.5601x; 1.4250x over previous
import functools
import math

import jax
import jax.numpy as jnp
from jax import lax
from jax.experimental import pallas as pl
from jax.experimental.pallas import tpu as pltpu
from jax.experimental.pallas import tpu_sc as plsc

NC = 2
NS = 16
NW = NC * NS

D_PAD = 16
DF = 32
KB = 5


def _sc_mesh():
    return plsc.VectorSubcoreMesh(
        core_axis_name="c", subcore_axis_name="s", num_cores=NC, num_subcores=NS
    )


_SC_PARAMS = pltpu.CompilerParams(use_tc_tiling_on_sc=False)


def _make_gather(n_chunk, ch):
    @functools.partial(
        pl.kernel,
        out_type=[
            jax.ShapeDtypeStruct((NW, n_chunk, ch, D_PAD), jnp.float32),
            jax.ShapeDtypeStruct((NW, n_chunk, ch, D_PAD), jnp.float32),
        ],
        mesh=_sc_mesh(),
        scratch_types=[
            pltpu.VMEM((KB, ch), jnp.int32),
            pltpu.VMEM((KB, ch), jnp.int32),
            pltpu.VMEM((KB, ch, D_PAD), jnp.float32),
            pltpu.VMEM((KB, ch, D_PAD), jnp.float32),
            pltpu.SemaphoreType.DMA,
            pltpu.SemaphoreType.DMA,
            pltpu.SemaphoreType.DMA,
        ],
        compiler_params=_SC_PARAMS,
    )
    def gather_k(y16, idx3, seg3, rep_out, self_out,
                 idx_v, seg_v, rep_v, self_v, sem_i, sem_g, sem_w):
        wid = lax.axis_index("s") * NC + lax.axis_index("c")

        @pl.loop(0, n_chunk, step=KB)
        def _body(j0):
            cps = []
            for b in range(KB):
                cps.append(pltpu.async_copy(idx3.at[wid, j0 + b],
                                            idx_v.at[b], sem_i))
                cps.append(pltpu.async_copy(seg3.at[wid, j0 + b],
                                            seg_v.at[b], sem_i))
            for cp in cps:
                cp.wait()
            gps = []
            for b in range(KB):
                gps.append(pltpu.async_copy(y16.at[idx_v.at[b]],
                                            rep_v.at[b], sem_g))
                gps.append(pltpu.async_copy(y16.at[seg_v.at[b]],
                                            self_v.at[b], sem_g))
            for cp in gps:
                cp.wait()
            wps = []
            for b in range(KB):
                wps.append(pltpu.async_copy(rep_v.at[b],
                                            rep_out.at[wid, j0 + b], sem_w))
                wps.append(pltpu.async_copy(self_v.at[b],
                                            self_out.at[wid, j0 + b], sem_w))
            for cp in wps:
                cp.wait()

    return gather_k


def _erf(x):
    a1, a2, a3, a4, a5 = (
        0.254829592, -0.284496736, 1.421413741, -1.453152027, 1.061405429
    )
    p = 0.3275911
    ax = jnp.abs(x)
    t = 1.0 / (1.0 + p * ax)
    poly = ((((a5 * t + a4) * t + a3) * t + a2) * t + a1) * t
    y = 1.0 - poly * jnp.exp(-ax * ax)
    return jnp.sign(x) * y


def _make_mlp(e, bt):
    grid = e // bt

    def mlp_body(rep_ref, self_ref, w1a_ref, w1b_ref, b1_ref, w2_ref, b2_ref,
                 out_ref):
        h = jnp.dot(rep_ref[...], w1a_ref[...],
                    preferred_element_type=jnp.float32)
        h = h + jnp.dot(self_ref[...], w1b_ref[...],
                        preferred_element_type=jnp.float32)
        h = h + b1_ref[...]
        inv_sqrt2 = 0.7071067811865476
        h = 0.5 * h * (1.0 + _erf(h * inv_sqrt2))
        out_ref[...] = (
            jnp.dot(h, w2_ref[...], preferred_element_type=jnp.float32)
            + b2_ref[...]
        )

    return pl.pallas_call(
        mlp_body,
        grid=(grid,),
        in_specs=[
            pl.BlockSpec((bt, D_PAD), lambda i: (i, 0)),
            pl.BlockSpec((bt, D_PAD), lambda i: (i, 0)),
            pl.BlockSpec((D_PAD, 64), lambda i: (0, 0)),
            pl.BlockSpec((D_PAD, 64), lambda i: (0, 0)),
            pl.BlockSpec((1, 64), lambda i: (0, 0)),
            pl.BlockSpec((64, DF), lambda i: (0, 0)),
            pl.BlockSpec((1, DF), lambda i: (0, 0)),
        ],
        out_specs=pl.BlockSpec((bt, DF), lambda i: (i, 0)),
        out_shape=jax.ShapeDtypeStruct((e, DF), jnp.float32),
    )


def _make_scatter(n, n_chunk, ch):
    rows_per_tile = n // NS

    @functools.partial(
        pl.kernel,
        out_type=jax.ShapeDtypeStruct((NC, n, DF), jnp.float32),
        mesh=_sc_mesh(),
        scratch_types=[
            pltpu.VMEM((KB, ch), jnp.int32),
            pltpu.VMEM((KB, ch), jnp.int32),
            pltpu.VMEM((KB, ch, DF), jnp.float32),
            pltpu.VMEM((KB, ch, DF), jnp.float32),
            pltpu.VMEM_SHARED((n, DF), jnp.float32),
            pltpu.SemaphoreType.DMA,
            pltpu.SemaphoreType.DMA,
        ],
        compiler_params=_SC_PARAMS,
    )
    def scatter_k(kout3, fy, idx3, seg3, zeros_hbm, out_hbm,
                  idx_v, seg_v, k_v, f_v, acc, sem_i, sem_g):
        cid = lax.axis_index("c")
        sid = lax.axis_index("s")
        wid = sid * NC + cid

        pltpu.sync_copy(
            zeros_hbm.at[pl.ds(sid * rows_per_tile, rows_per_tile)],
            acc.at[pl.ds(sid * rows_per_tile, rows_per_tile)],
        )
        plsc.subcore_barrier()

        @pl.loop(0, n_chunk, step=KB)
        def _body(j0):
            cps = []
            for b in range(KB):
                cps.append(pltpu.async_copy(idx3.at[wid, j0 + b],
                                            idx_v.at[b], sem_i))
                cps.append(pltpu.async_copy(seg3.at[wid, j0 + b],
                                            seg_v.at[b], sem_i))
                cps.append(pltpu.async_copy(kout3.at[wid, j0 + b],
                                            k_v.at[b], sem_i))
            for cp in cps:
                cp.wait()
            gps = []
            for b in range(KB):
                gps.append(pltpu.async_copy(fy.at[idx_v.at[b]],
                                            f_v.at[b], sem_g))
            for cp in gps:
                cp.wait()
            for b in range(KB):
                @pl.loop(0, ch)
                def _mul(r, _b=b):
                    for hh in range(DF // 16):
                        s = pl.ds(hh * 16, 16)
                        k_v[_b, r, s] = k_v[_b, r, s] * f_v[_b, r, s]
            for b in range(KB):
                pltpu.sync_copy(k_v.at[b], acc.at[seg_v.at[b]], add=True)

        plsc.subcore_barrier()
        pltpu.sync_copy(
            acc.at[pl.ds(sid * rows_per_tile, rows_per_tile)],
            out_hbm.at[cid, pl.ds(sid * rows_per_tile, rows_per_tile)],
        )

    return scatter_k


def _make_combine(n, bn):
    grid = n // bn

    def comb_body(p_ref, inv_ref, out_ref):
        out_ref[...] = (p_ref[0] + p_ref[1]) * inv_ref[...]

    return pl.pallas_call(
        comb_body,
        grid=(grid,),
        in_specs=[
            pl.BlockSpec((NC, bn, DF), lambda i: (0, i, 0)),
            pl.BlockSpec((bn, 1), lambda i: (i, 0)),
        ],
        out_specs=pl.BlockSpec((bn, DF), lambda i: (i, 0)),
        out_shape=jax.ShapeDtypeStruct((n, DF), jnp.float32),
    )


def kernel(y, f_y, neighbors_index, neighbors_row_splits, W1, b1, W2, b2):
    n, d_coord = y.shape
    e = neighbors_index.shape[0]
    assert e % NW == 0
    epw = e // NW
    ch = 128
    while epw % ch != 0 or ch % 8 != 0 or (epw // ch) % KB != 0:
        ch -= 8
    n_chunk = epw // ch

    idx = neighbors_index.astype(jnp.int32)
    rs = neighbors_row_splits.astype(jnp.int32)
    hist = jnp.zeros((e,), jnp.int32).at[rs[1:n]].add(1, mode="drop")
    seg = jnp.clip(jnp.cumsum(hist), 0, n - 1)

    idx3 = idx.reshape(NW, n_chunk, ch)
    seg3 = seg.reshape(NW, n_chunk, ch)

    y16 = jnp.zeros((n, D_PAD), jnp.float32).at[:, :d_coord].set(y)

    rep16, self16 = _make_gather(n_chunk, ch)(y16, idx3, seg3)
    rep2 = rep16.reshape(e, D_PAD)
    self2 = self16.reshape(e, D_PAD)

    w1a = jnp.zeros((D_PAD, 64), jnp.float32).at[:d_coord].set(W1[:d_coord])
    w1b = jnp.zeros((D_PAD, 64), jnp.float32).at[:d_coord].set(W1[d_coord:])

    kout = _make_mlp(e, 4000)(rep2, self2, w1a, w1b,
                              b1.reshape(1, 64), W2, b2.reshape(1, DF))
    kout3 = kout.reshape(NW, n_chunk, ch, DF)

    zeros_hbm = jnp.zeros((n, DF), jnp.float32)
    partials = _make_scatter(n, n_chunk, ch)(kout3, f_y, idx3, seg3, zeros_hbm)

    counts = (rs[1:] - rs[:-1]).astype(jnp.float32)
    inv = (1.0 / jnp.maximum(counts, 1.0)).reshape(n, 1)

    bn = 2000
    while n % bn != 0:
        bn -= 8
    out = _make_combine(n, bn)(partials, inv)
    return out
